# baseline (speedup 1.0000x reference)
import functools

import jax
import jax.numpy as jnp
from jax.experimental import pallas as pl
from jax.experimental.pallas import tpu as pltpu

_NSTREAM = 2
_NBLOCKS = 4


def _accumulate(adj_refs, op_ref, acc_ref, k, *, n, bm, bk, nsteps):
    ns = len(adj_refs)
    for s, a_ref in enumerate(adj_refs):
        base = s * nsteps * bk
        if ns > 1 and s == 0:
            @pl.when(k == 0)
            def _first():
                acc_ref[...] = jnp.dot(
                    a_ref[...].astype(jnp.bfloat16),
                    op_ref[pl.ds(base, bk), :],
                    preferred_element_type=jnp.float32)

            @pl.when(k > 0)
            def _rest():
                acc_ref[...] += jnp.dot(
                    a_ref[...].astype(jnp.bfloat16),
                    op_ref[pl.ds(k * bk + base, bk), :],
                    preferred_element_type=jnp.float32)
        elif s < ns - 1:
            acc_ref[...] += jnp.dot(
                a_ref[...].astype(jnp.bfloat16),
                op_ref[pl.ds(k * bk + base, bk), :],
                preferred_element_type=jnp.float32)
        else:
            w = n - (ns - 1) * nsteps * bk - (nsteps - 1) * bk
            remfull = (w // 128) * 128
            tailw = w - remfull

            @pl.when(k < nsteps - 1)
            def _full():
                acc_ref[...] += jnp.dot(
                    a_ref[...].astype(jnp.bfloat16),
                    op_ref[pl.ds(k * bk + base, bk), :],
                    preferred_element_type=jnp.float32)

            @pl.when(k == nsteps - 1)
            def _ragged():
                kb = (nsteps - 1) * bk + base
                if remfull > 0:
                    acc_ref[...] += jnp.dot(
                        a_ref[:, :remfull].astype(jnp.bfloat16),
                        op_ref[pl.ds(kb, remfull), :],
                        preferred_element_type=jnp.float32)
                if tailw > 0:
                    at = a_ref[:, remfull:remfull + 128]
                    col = jax.lax.broadcasted_iota(jnp.int32, (bm, 128), 1)
                    at = jnp.where(col < tailw, at, 0.0).astype(jnp.bfloat16)
                    acc_ref[...] += jnp.dot(
                        at, op_ref[pl.ds(kb + remfull, 128), :],
                        preferred_element_type=jnp.float32)


def _gcn_kernel(*refs, n, bm, bk, nsteps, npad):
    ns = _NSTREAM
    adj_refs = refs[:ns]
    (x_ref, w1_ref, b1_ref, w2_ref, b2_ref, out_ref,
     y_ref, g_ref, acc1_ref, acc2_ref) = refs[ns:]
    p = pl.program_id(0)
    i = pl.program_id(1)
    k = pl.program_id(2)

    @pl.when((p == 0) & (i == 0) & (k == 0))
    def _compute_y():
        y = jnp.dot(x_ref[...], w1_ref[...],
                    preferred_element_type=jnp.float32)
        y_ref[:n, :] = y.astype(jnp.bfloat16)
        if npad > n:
            y_ref[n:, :] = jnp.zeros((npad - n, y_ref.shape[1]),
                                     jnp.bfloat16)

    @pl.when(p == 0)
    def _phase0():
        _accumulate(adj_refs, y_ref, acc1_ref, k, n=n, bm=bm, bk=bk,
                    nsteps=nsteps)

        @pl.when(k == nsteps - 1)
        def _epilogue():
            h = jnp.maximum(acc1_ref[...] + b1_ref[...],
                            0.0).astype(jnp.bfloat16)
            g = jnp.dot(h, w2_ref[...], preferred_element_type=jnp.float32)
            row = jax.lax.broadcasted_iota(jnp.int32, g.shape, 0)
            g = jnp.where(row + i * bm < n, g, 0.0)
            g_ref[pl.ds(i * bm, bm), :] = g.astype(jnp.bfloat16)

    @pl.when(p == 1)
    def _phase1():
        _accumulate(adj_refs, g_ref, acc2_ref, k, n=n, bm=bm, bk=bk,
                    nsteps=nsteps)

        @pl.when(k == nsteps - 1)
        def _epilogue():
            o = acc2_ref[...] + b2_ref[...]
            m = jnp.max(o, axis=1, keepdims=True)
            s = o - m
            lse = jnp.log(jnp.sum(jnp.exp(s), axis=1, keepdims=True))
            out_ref[...] = s - lse


def kernel(x, adj, W1, b1, W2, b2):
    n, nfeat = x.shape
    nhid = W1.shape[1]
    nclass = W2.shape[1]

    bm = 704
    bk = (((n + _NBLOCKS - 1) // _NBLOCKS) + 127) // 128 * 128
    nsteps = _NBLOCKS // _NSTREAM
    ni = (n + bm - 1) // bm
    npad = _NBLOCKS * bk
    gpad = max(npad, ni * bm)

    b1r = b1.reshape(1, nhid)
    b2r = b2.reshape(1, nclass)
    w2b = W2.astype(jnp.bfloat16)

    def _adj_spec(s):
        return pl.BlockSpec((bm, bk),
                            lambda p, i, k, s=s: (i, k + s * nsteps))

    out = pl.pallas_call(
        functools.partial(_gcn_kernel, n=n, bm=bm, bk=bk, nsteps=nsteps,
                          npad=npad),
        grid=(2, ni, nsteps),
        in_specs=[_adj_spec(s) for s in range(_NSTREAM)] + [
            pl.BlockSpec((n, nfeat), lambda p, i, k: (0, 0)),
            pl.BlockSpec((nfeat, nhid), lambda p, i, k: (0, 0)),
            pl.BlockSpec((1, nhid), lambda p, i, k: (0, 0)),
            pl.BlockSpec((nhid, nclass), lambda p, i, k: (0, 0)),
            pl.BlockSpec((1, nclass), lambda p, i, k: (0, 0)),
        ],
        out_specs=pl.BlockSpec(
            (bm, nclass), lambda p, i, k: (jnp.where(p == 1, i, 0), 0)),
        out_shape=jax.ShapeDtypeStruct((n, nclass), jnp.float32),
        scratch_shapes=[
            pltpu.VMEM((npad, nhid), jnp.bfloat16),
            pltpu.VMEM((gpad, nclass), jnp.bfloat16),
            pltpu.VMEM((bm, nhid), jnp.float32),
            pltpu.VMEM((bm, nclass), jnp.float32),
        ],
        compiler_params=pltpu.CompilerParams(
            dimension_semantics=("arbitrary", "arbitrary", "arbitrary")),
    )(*([adj] * _NSTREAM), x, W1, b1r, w2b, b2r)

    return out


# speedup vs baseline: 1.0164x; 1.0164x over previous
import functools

import jax
import jax.numpy as jnp
from jax.experimental import pallas as pl
from jax.experimental.pallas import tpu as pltpu

_NSTREAM = 2
_NBLOCKS = 4


def _accumulate(adj_refs, op_ref, acc_ref, k, *, n, bm, bk, nsteps):
    ns = len(adj_refs)
    for s, a_ref in enumerate(adj_refs):
        base = s * nsteps * bk
        if ns > 1 and s == 0:
            @pl.when(k == 0)
            def _first():
                acc_ref[...] = jnp.dot(
                    a_ref[...].astype(jnp.bfloat16),
                    op_ref[pl.ds(base, bk), :],
                    preferred_element_type=jnp.float32)

            @pl.when(k > 0)
            def _rest():
                acc_ref[...] += jnp.dot(
                    a_ref[...].astype(jnp.bfloat16),
                    op_ref[pl.ds(k * bk + base, bk), :],
                    preferred_element_type=jnp.float32)
        elif s < ns - 1:
            acc_ref[...] += jnp.dot(
                a_ref[...].astype(jnp.bfloat16),
                op_ref[pl.ds(k * bk + base, bk), :],
                preferred_element_type=jnp.float32)
        else:
            w = n - (ns - 1) * nsteps * bk - (nsteps - 1) * bk
            remfull = (w // 128) * 128
            tailw = w - remfull

            @pl.when(k < nsteps - 1)
            def _full():
                acc_ref[...] += jnp.dot(
                    a_ref[...].astype(jnp.bfloat16),
                    op_ref[pl.ds(k * bk + base, bk), :],
                    preferred_element_type=jnp.float32)

            @pl.when(k == nsteps - 1)
            def _ragged():
                kb = (nsteps - 1) * bk + base
                if remfull > 0:
                    acc_ref[...] += jnp.dot(
                        a_ref[:, :remfull].astype(jnp.bfloat16),
                        op_ref[pl.ds(kb, remfull), :],
                        preferred_element_type=jnp.float32)
                if tailw > 0:
                    at = a_ref[:, remfull:remfull + 128]
                    col = jax.lax.broadcasted_iota(jnp.int32, (bm, 128), 1)
                    at = jnp.where(col < tailw, at, 0.0).astype(jnp.bfloat16)
                    acc_ref[...] += jnp.dot(
                        at, op_ref[pl.ds(kb + remfull, 128), :],
                        preferred_element_type=jnp.float32)


def _gcn_kernel(*refs, n, bm, bk, nsteps, npad):
    ns = _NSTREAM
    adj_refs = refs[:ns]
    (x_ref, w1_ref, b1_ref, w2_ref, b2_ref, out_ref,
     y_ref, g_ref, acc1_ref, acc2_ref) = refs[ns:]
    p = pl.program_id(0)
    i = pl.program_id(1)
    k = pl.program_id(2)

    @pl.when((p == 0) & (i == 0) & (k == 0))
    def _compute_y():
        y = jnp.dot(x_ref[...], w1_ref[...],
                    preferred_element_type=jnp.float32)
        y_ref[:n, :] = y.astype(jnp.bfloat16)
        if npad > n:
            y_ref[n:, :] = jnp.zeros((npad - n, y_ref.shape[1]),
                                     jnp.bfloat16)

    @pl.when(p == 0)
    def _phase0():
        _accumulate(adj_refs, y_ref, acc1_ref, k, n=n, bm=bm, bk=bk,
                    nsteps=nsteps)

        @pl.when(k == nsteps - 1)
        def _epilogue():
            h = jnp.maximum(acc1_ref[...] + b1_ref[...],
                            0.0).astype(jnp.bfloat16)
            g = jnp.dot(h, w2_ref[...], preferred_element_type=jnp.float32)
            row = jax.lax.broadcasted_iota(jnp.int32, g.shape, 0)
            g = jnp.where(row + i * bm < n, g, 0.0)
            g_ref[pl.ds(i * bm, bm), :] = g.astype(jnp.bfloat16)

    @pl.when(p == 1)
    def _phase1():
        _accumulate(adj_refs, g_ref, acc2_ref, k, n=n, bm=bm, bk=bk,
                    nsteps=nsteps)

        @pl.when(k == nsteps - 1)
        def _epilogue():
            o = acc2_ref[...] + b2_ref[...]
            m = jnp.max(o, axis=1, keepdims=True)
            s = o - m
            lse = jnp.log(jnp.sum(jnp.exp(s), axis=1, keepdims=True))
            out_ref[...] = s - lse


def kernel(x, adj, W1, b1, W2, b2):
    n, nfeat = x.shape
    nhid = W1.shape[1]
    nclass = W2.shape[1]

    bm = 576
    bk = (((n + _NBLOCKS - 1) // _NBLOCKS) + 127) // 128 * 128
    nsteps = _NBLOCKS // _NSTREAM
    ni = (n + bm - 1) // bm
    npad = _NBLOCKS * bk
    gpad = max(npad, ni * bm)

    b1r = b1.reshape(1, nhid)
    b2r = b2.reshape(1, nclass)
    w2b = W2.astype(jnp.bfloat16)

    def _adj_spec(s):
        return pl.BlockSpec((bm, bk),
                            lambda p, i, k, s=s: (i, k + s * nsteps))

    out = pl.pallas_call(
        functools.partial(_gcn_kernel, n=n, bm=bm, bk=bk, nsteps=nsteps,
                          npad=npad),
        grid=(2, ni, nsteps),
        in_specs=[_adj_spec(s) for s in range(_NSTREAM)] + [
            pl.BlockSpec((n, nfeat), lambda p, i, k: (0, 0)),
            pl.BlockSpec((nfeat, nhid), lambda p, i, k: (0, 0)),
            pl.BlockSpec((1, nhid), lambda p, i, k: (0, 0)),
            pl.BlockSpec((nhid, nclass), lambda p, i, k: (0, 0)),
            pl.BlockSpec((1, nclass), lambda p, i, k: (0, 0)),
        ],
        out_specs=pl.BlockSpec(
            (bm, nclass), lambda p, i, k: (jnp.where(p == 1, i, 0), 0)),
        out_shape=jax.ShapeDtypeStruct((n, nclass), jnp.float32),
        scratch_shapes=[
            pltpu.VMEM((npad, nhid), jnp.bfloat16),
            pltpu.VMEM((gpad, nclass), jnp.bfloat16),
            pltpu.VMEM((bm, nhid), jnp.float32),
            pltpu.VMEM((bm, nclass), jnp.float32),
        ],
        compiler_params=pltpu.CompilerParams(
            dimension_semantics=("arbitrary", "arbitrary", "arbitrary")),
    )(*([adj] * _NSTREAM), x, W1, b1r, w2b, b2r)

    return out


# speedup vs baseline: 1.0250x; 1.0084x over previous
import functools

import jax
import jax.numpy as jnp
from jax.experimental import pallas as pl
from jax.experimental.pallas import tpu as pltpu

_NSTREAM = 2
_NBLOCKS = 4


def _accumulate(adj_refs, op_ref, acc_ref, k, *, n, bm, bk, nsteps):
    ns = len(adj_refs)
    for s, a_ref in enumerate(adj_refs):
        base = s * nsteps * bk
        if ns > 1 and s == 0:
            @pl.when(k == 0)
            def _first():
                acc_ref[...] = jnp.dot(
                    a_ref[...].astype(jnp.bfloat16),
                    op_ref[pl.ds(base, bk), :],
                    preferred_element_type=jnp.float32)

            @pl.when(k > 0)
            def _rest():
                acc_ref[...] += jnp.dot(
                    a_ref[...].astype(jnp.bfloat16),
                    op_ref[pl.ds(k * bk + base, bk), :],
                    preferred_element_type=jnp.float32)
        elif s < ns - 1:
            acc_ref[...] += jnp.dot(
                a_ref[...].astype(jnp.bfloat16),
                op_ref[pl.ds(k * bk + base, bk), :],
                preferred_element_type=jnp.float32)
        else:
            w = n - (ns - 1) * nsteps * bk - (nsteps - 1) * bk
            remfull = (w // 128) * 128
            tailw = w - remfull

            @pl.when(k < nsteps - 1)
            def _full():
                acc_ref[...] += jnp.dot(
                    a_ref[...].astype(jnp.bfloat16),
                    op_ref[pl.ds(k * bk + base, bk), :],
                    preferred_element_type=jnp.float32)

            @pl.when(k == nsteps - 1)
            def _ragged():
                kb = (nsteps - 1) * bk + base
                if remfull > 0:
                    acc_ref[...] += jnp.dot(
                        a_ref[:, :remfull].astype(jnp.bfloat16),
                        op_ref[pl.ds(kb, remfull), :],
                        preferred_element_type=jnp.float32)
                if tailw > 0:
                    at = a_ref[:, remfull:remfull + 128]
                    col = jax.lax.broadcasted_iota(jnp.int32, (bm, 128), 1)
                    at = jnp.where(col < tailw, at, 0.0).astype(jnp.bfloat16)
                    acc_ref[...] += jnp.dot(
                        at, op_ref[pl.ds(kb + remfull, 128), :],
                        preferred_element_type=jnp.float32)


def _gcn_kernel(*refs, n, bm, bk, nsteps, npad):
    ns = _NSTREAM
    adj_refs = refs[:ns]
    (x_ref, w1_ref, b1_ref, w2_ref, b2_ref, out_ref,
     y_ref, g_ref, acc1_ref, acc2_ref) = refs[ns:]
    p = pl.program_id(0)
    i = pl.program_id(1)
    k = pl.program_id(2)

    @pl.when((p == 0) & (i == 0) & (k == 0))
    def _compute_y():
        y = jnp.dot(x_ref[...], w1_ref[...],
                    preferred_element_type=jnp.float32)
        y_ref[:n, :] = y.astype(jnp.bfloat16)
        if npad > n:
            y_ref[n:, :] = jnp.zeros((npad - n, y_ref.shape[1]),
                                     jnp.bfloat16)

    @pl.when(p == 0)
    def _phase0():
        _accumulate(adj_refs, y_ref, acc1_ref, k, n=n, bm=bm, bk=bk,
                    nsteps=nsteps)

        @pl.when(k == nsteps - 1)
        def _epilogue():
            h = jnp.maximum(acc1_ref[...] + b1_ref[...],
                            0.0).astype(jnp.bfloat16)
            g = jnp.dot(h, w2_ref[...], preferred_element_type=jnp.float32)
            row = jax.lax.broadcasted_iota(jnp.int32, g.shape, 0)
            g = jnp.where(row + i * bm < n, g, 0.0)
            g_ref[pl.ds(i * bm, bm), :] = g.astype(jnp.bfloat16)

    @pl.when(p == 1)
    def _phase1():
        _accumulate(adj_refs, g_ref, acc2_ref, k, n=n, bm=bm, bk=bk,
                    nsteps=nsteps)

        @pl.when(k == nsteps - 1)
        def _epilogue():
            o = acc2_ref[...] + b2_ref[...]
            m = jnp.max(o, axis=1, keepdims=True)
            s = o - m
            lse = jnp.log(jnp.sum(jnp.exp(s), axis=1, keepdims=True))
            out_ref[...] = s - lse


def kernel(x, adj, W1, b1, W2, b2):
    n, nfeat = x.shape
    nhid = W1.shape[1]
    nclass = W2.shape[1]

    bm = 640
    bk = (((n + _NBLOCKS - 1) // _NBLOCKS) + 127) // 128 * 128
    nsteps = _NBLOCKS // _NSTREAM
    ni = (n + bm - 1) // bm
    npad = _NBLOCKS * bk
    gpad = max(npad, ni * bm)

    b1r = b1.reshape(1, nhid)
    b2r = b2.reshape(1, nclass)
    w2b = W2.astype(jnp.bfloat16)

    def _adj_spec(s):
        return pl.BlockSpec((bm, bk),
                            lambda p, i, k, s=s: (i, k + s * nsteps))

    out = pl.pallas_call(
        functools.partial(_gcn_kernel, n=n, bm=bm, bk=bk, nsteps=nsteps,
                          npad=npad),
        grid=(2, ni, nsteps),
        in_specs=[_adj_spec(s) for s in range(_NSTREAM)] + [
            pl.BlockSpec((n, nfeat), lambda p, i, k: (0, 0)),
            pl.BlockSpec((nfeat, nhid), lambda p, i, k: (0, 0)),
            pl.BlockSpec((1, nhid), lambda p, i, k: (0, 0)),
            pl.BlockSpec((nhid, nclass), lambda p, i, k: (0, 0)),
            pl.BlockSpec((1, nclass), lambda p, i, k: (0, 0)),
        ],
        out_specs=pl.BlockSpec(
            (bm, nclass), lambda p, i, k: (jnp.where(p == 1, i, 0), 0)),
        out_shape=jax.ShapeDtypeStruct((n, nclass), jnp.float32),
        scratch_shapes=[
            pltpu.VMEM((npad, nhid), jnp.bfloat16),
            pltpu.VMEM((gpad, nclass), jnp.bfloat16),
            pltpu.VMEM((bm, nhid), jnp.float32),
            pltpu.VMEM((bm, nclass), jnp.float32),
        ],
        compiler_params=pltpu.CompilerParams(
            dimension_semantics=("arbitrary", "arbitrary", "arbitrary")),
    )(*([adj] * _NSTREAM), x, W1, b1r, w2b, b2r)

    return out


# speedup vs baseline: 1.0307x; 1.0055x over previous
import functools

import jax
import jax.numpy as jnp
from jax.experimental import pallas as pl
from jax.experimental.pallas import tpu as pltpu

_NSTREAM = 2
_NBLOCKS = 4


def _accumulate(adj_refs, op_ref, acc_ref, k, *, n, bm, bk, nsteps):
    ns = len(adj_refs)
    for s, a_ref in enumerate(adj_refs):
        base = s * nsteps * bk
        if ns > 1 and s == 0:
            @pl.when(k == 0)
            def _first():
                acc_ref[...] = jnp.dot(
                    a_ref[...].astype(jnp.bfloat16),
                    op_ref[pl.ds(base, bk), :],
                    preferred_element_type=jnp.float32)

            @pl.when(k > 0)
            def _rest():
                acc_ref[...] += jnp.dot(
                    a_ref[...].astype(jnp.bfloat16),
                    op_ref[pl.ds(k * bk + base, bk), :],
                    preferred_element_type=jnp.float32)
        elif s < ns - 1:
            acc_ref[...] += jnp.dot(
                a_ref[...].astype(jnp.bfloat16),
                op_ref[pl.ds(k * bk + base, bk), :],
                preferred_element_type=jnp.float32)
        else:
            w = n - (ns - 1) * nsteps * bk - (nsteps - 1) * bk
            remfull = (w // 128) * 128
            tailw = w - remfull

            @pl.when(k < nsteps - 1)
            def _full():
                acc_ref[...] += jnp.dot(
                    a_ref[...].astype(jnp.bfloat16),
                    op_ref[pl.ds(k * bk + base, bk), :],
                    preferred_element_type=jnp.float32)

            @pl.when(k == nsteps - 1)
            def _ragged():
                kb = (nsteps - 1) * bk + base
                if remfull > 0:
                    acc_ref[...] += jnp.dot(
                        a_ref[:, :remfull].astype(jnp.bfloat16),
                        op_ref[pl.ds(kb, remfull), :],
                        preferred_element_type=jnp.float32)
                if tailw > 0:
                    at = a_ref[:, remfull:remfull + 128]
                    col = jax.lax.broadcasted_iota(jnp.int32, (bm, 128), 1)
                    at = jnp.where(col < tailw, at, 0.0).astype(jnp.bfloat16)
                    acc_ref[...] += jnp.dot(
                        at, op_ref[pl.ds(kb + remfull, 128), :],
                        preferred_element_type=jnp.float32)


def _gcn_kernel(*refs, n, bm, bk, nsteps, npad):
    ns = _NSTREAM
    adj_refs = refs[:ns]
    (x_ref, w1_ref, b1_ref, w2_ref, b2_ref, out_ref,
     y_ref, g_ref, acc1_ref, acc2_ref) = refs[ns:]
    p = pl.program_id(0)
    i = pl.program_id(1)
    k = pl.program_id(2)

    @pl.when((p == 0) & (i == 0) & (k == 0))
    def _compute_y():
        y = jnp.dot(x_ref[...], w1_ref[...],
                    preferred_element_type=jnp.float32)
        y_ref[:n, :] = y.astype(jnp.bfloat16)
        if npad > n:
            y_ref[n:, :] = jnp.zeros((npad - n, y_ref.shape[1]),
                                     jnp.bfloat16)

    @pl.when(p == 0)
    def _phase0():
        _accumulate(adj_refs, y_ref, acc1_ref, k, n=n, bm=bm, bk=bk,
                    nsteps=nsteps)

        @pl.when(k == nsteps - 1)
        def _epilogue():
            h = jnp.maximum(acc1_ref[...] + b1_ref[...],
                            0.0).astype(jnp.bfloat16)
            g = jnp.dot(h, w2_ref[...], preferred_element_type=jnp.float32)
            row = jax.lax.broadcasted_iota(jnp.int32, g.shape, 0)
            g = jnp.where(row + i * bm < n, g, 0.0)
            g_ref[pl.ds(i * bm, bm), :] = g.astype(jnp.bfloat16)

    @pl.when(p == 1)
    def _phase1():
        _accumulate(adj_refs, g_ref, acc2_ref, k, n=n, bm=bm, bk=bk,
                    nsteps=nsteps)

        @pl.when(k == nsteps - 1)
        def _epilogue():
            o = acc2_ref[...] + b2_ref[...]
            m = jnp.max(o, axis=1, keepdims=True)
            s = o - m
            lse = jnp.log(jnp.sum(jnp.exp(s), axis=1, keepdims=True))
            out_ref[...] = s - lse


def kernel(x, adj, W1, b1, W2, b2):
    n, nfeat = x.shape
    nhid = W1.shape[1]
    nclass = W2.shape[1]

    bm = 672
    bk = (((n + _NBLOCKS - 1) // _NBLOCKS) + 127) // 128 * 128
    nsteps = _NBLOCKS // _NSTREAM
    ni = (n + bm - 1) // bm
    npad = _NBLOCKS * bk
    gpad = max(npad, ni * bm)

    b1r = b1.reshape(1, nhid)
    b2r = b2.reshape(1, nclass)
    w2b = W2.astype(jnp.bfloat16)

    def _adj_spec(s):
        return pl.BlockSpec((bm, bk),
                            lambda p, i, k, s=s: (i, k + s * nsteps))

    out = pl.pallas_call(
        functools.partial(_gcn_kernel, n=n, bm=bm, bk=bk, nsteps=nsteps,
                          npad=npad),
        grid=(2, ni, nsteps),
        in_specs=[_adj_spec(s) for s in range(_NSTREAM)] + [
            pl.BlockSpec((n, nfeat), lambda p, i, k: (0, 0)),
            pl.BlockSpec((nfeat, nhid), lambda p, i, k: (0, 0)),
            pl.BlockSpec((1, nhid), lambda p, i, k: (0, 0)),
            pl.BlockSpec((nhid, nclass), lambda p, i, k: (0, 0)),
            pl.BlockSpec((1, nclass), lambda p, i, k: (0, 0)),
        ],
        out_specs=pl.BlockSpec(
            (bm, nclass), lambda p, i, k: (jnp.where(p == 1, i, 0), 0)),
        out_shape=jax.ShapeDtypeStruct((n, nclass), jnp.float32),
        scratch_shapes=[
            pltpu.VMEM((npad, nhid), jnp.bfloat16),
            pltpu.VMEM((gpad, nclass), jnp.bfloat16),
            pltpu.VMEM((bm, nhid), jnp.float32),
            pltpu.VMEM((bm, nclass), jnp.float32),
        ],
        compiler_params=pltpu.CompilerParams(
            dimension_semantics=("arbitrary", "arbitrary", "arbitrary")),
    )(*([adj] * _NSTREAM), x, W1, b1r, w2b, b2r)

    return out
